# Initial kernel scaffold; baseline (speedup 1.0000x reference)
import jax
import jax.numpy as jnp
from jax.experimental import pallas as pl


def kernel(pred, gt, train_mask):
    raise NotImplementedError("write your pallas kernel here")


# speedup vs baseline: 76.6106x; 76.6106x over previous
import jax
import jax.numpy as jnp
from jax import lax
from jax.experimental import pallas as pl
from jax.experimental.pallas import tpu as pltpu

_NEG_RATIO = 3.0
_EPS = 1e-4

_R = 4096
_C = 1024
_GRID = 8
_BR = _R // _GRID


def _loss_terms(pred, g, mask):
    p = jnp.where(g > 0.5, pred, 1.0 - pred)
    loss = -jnp.log(p)
    ml = loss * mask
    pos = g * mask
    return ml, pos


def _reduce_kernel(pred_ref, gt_ref, mask_ref, out_ref):
    i = pl.program_id(0)
    pred = pred_ref[...]
    g = gt_ref[...]
    mask = mask_ref[...]
    ml, pos = _loss_terms(pred, g, mask)
    pos_sum = jnp.sum(ml * g)
    neg_sum = jnp.sum(ml * (1.0 - g))
    pcnt = jnp.sum(pos)
    mcnt = jnp.sum(mask)

    @pl.when(i == 0)
    def _():
        out_ref[0] = 0.0
        out_ref[1] = 0.0
        out_ref[2] = 0.0
        out_ref[3] = 0.0

    out_ref[0] += pos_sum
    out_ref[1] += neg_sum
    out_ref[2] += pcnt
    out_ref[3] += mcnt - pcnt


def _select_kernel(th_ref, pred_ref, gt_ref, mask_ref, out_ref):
    i = pl.program_id(0)
    t = th_ref[0]
    pred = pred_ref[...]
    g = gt_ref[...]
    mask = mask_ref[...]
    ml, _ = _loss_terms(pred, g, mask)
    nl = ml * (1.0 - g)
    bits = lax.bitcast_convert_type(nl, jnp.int32)
    ge = (bits >= t).astype(jnp.float32)
    gt_m = bits > t
    gtf = gt_m.astype(jnp.float32)

    @pl.when(i == 0)
    def _():
        out_ref[0] = 0.0
        out_ref[1] = 0.0
        out_ref[2] = 0.0

    out_ref[0] += jnp.sum(ge)
    out_ref[1] += jnp.sum(gtf)
    out_ref[2] += jnp.sum(jnp.where(gt_m, nl, 0.0))


def _in_specs(n):
    return [pl.BlockSpec((_BR, _C), lambda i: (i, 0)) for _ in range(n)]


def _run_reduce(p2, g2, m2):
    return pl.pallas_call(
        _reduce_kernel,
        grid=(_GRID,),
        in_specs=_in_specs(3),
        out_specs=pl.BlockSpec(memory_space=pltpu.SMEM),
        out_shape=jax.ShapeDtypeStruct((4,), jnp.float32),
    )(p2, g2, m2)


def _run_select(th, p2, g2, m2):
    return pl.pallas_call(
        _select_kernel,
        grid=(_GRID,),
        in_specs=[pl.BlockSpec(memory_space=pltpu.SMEM)] + _in_specs(3),
        out_specs=pl.BlockSpec(memory_space=pltpu.SMEM),
        out_shape=jax.ShapeDtypeStruct((3,), jnp.float32),
    )(th, p2, g2, m2)


def kernel(pred, gt, train_mask):
    p2 = pred.reshape(_R, _C)
    g2 = gt.reshape(_R, _C)
    m2 = train_mask.reshape(_R, _C)

    sums = _run_reduce(p2, g2, m2)
    pos_sum, neg_sum, pcnt, ncnt = sums[0], sums[1], sums[2], sums[3]
    k = jnp.minimum(ncnt, jnp.floor(pcnt * _NEG_RATIO))

    def common(_):
        return neg_sum

    def rare(_):
        def body(i, prefix):
            cand = prefix | (1 << (30 - i))
            s = _run_select(cand[None], p2, g2, m2)
            return jnp.where(s[0] >= k, cand, prefix)

        t = lax.fori_loop(0, 31, body, jnp.int32(0))
        s = _run_select(t[None], p2, g2, m2)
        tval = lax.bitcast_convert_type(t, jnp.float32)
        extra = k - s[1]
        return s[2] + jnp.where(extra > 0, extra * tval, 0.0)

    neg_sel = lax.cond(k >= ncnt, common, rare, None)
    return (pos_sum + neg_sel) / (pcnt + k + _EPS)
